# baseline (device time: 58840 ns/iter reference)
import jax
import jax.numpy as jnp
from jax import lax
from jax.experimental import pallas as pl
from jax.experimental.pallas import tpu as pltpu

N_DEV = 32
M = 768
D = 768
CHUNK = M // N_DEV
NBLK = 4
RPB = M // NBLK
CPB = N_DEV // NBLK


def kernel(x, Wg, Wu, Wd):
    def body(x_ref, wg_ref, wu_ref, wd_ref, out_ref,
             partial_ref, comm_ref, gather_ref,
             send1_sems, recv1_sems, send2_sems, recv2_sems):
        me = lax.axis_index("i")

        barrier = pltpu.get_barrier_semaphore()
        for d in range(1, N_DEV):
            peer = lax.rem(me + d, N_DEV)
            pl.semaphore_signal(barrier, inc=1, device_id=(peer,),
                                device_id_type=pl.DeviceIdType.MESH)
        pl.semaphore_wait(barrier, N_DEV - 1)

        wg = wg_ref[...].astype(jnp.bfloat16)
        wu = wu_ref[...].astype(jnp.bfloat16)
        wd = wd_ref[...].astype(jnp.bfloat16)

        sends1 = []
        for b in range(NBLK):
            r0 = b * RPB
            xb = x_ref[r0:r0 + RPB, :].astype(jnp.bfloat16)
            gate = jnp.dot(xb, wg, preferred_element_type=jnp.float32)
            up = jnp.dot(xb, wu, preferred_element_type=jnp.float32)
            act = (gate * (up * jax.nn.sigmoid(up))).astype(jnp.bfloat16)
            pblk = jnp.dot(act, wd, preferred_element_type=jnp.float32
                           ).astype(jnp.bfloat16)
            for j in range(CPB):
                c = b * CPB + j
                partial_ref[c] = pblk[j * CHUNK:(j + 1) * CHUNK, :]
            for j in range(CPB):
                c = b * CPB + j
                rdma = pltpu.make_async_remote_copy(
                    src_ref=partial_ref.at[c],
                    dst_ref=comm_ref.at[me],
                    send_sem=send1_sems.at[c],
                    recv_sem=recv1_sems.at[me],
                    device_id=(c,),
                    device_id_type=pl.DeviceIdType.MESH,
                )
                pl.when(c != me)(rdma.start)
                sends1.append((c, rdma))

        comm_ref[pl.ds(me, 1)] = partial_ref[pl.ds(me, 1)]

        for c in range(N_DEV):
            rdma = pltpu.make_async_remote_copy(
                src_ref=partial_ref.at[c],
                dst_ref=comm_ref.at[c],
                send_sem=send1_sems.at[c],
                recv_sem=recv1_sems.at[c],
                device_id=(c,),
                device_id_type=pl.DeviceIdType.MESH,
            )
            pl.when(c != me)(rdma.wait_recv)
        for c, rdma in sends1:
            pl.when(c != me)(rdma.wait_send)

        reduced = jnp.sum(comm_ref[...].astype(jnp.float32), axis=0)
        gather_ref[pl.ds(me, 1)] = reduced.astype(jnp.bfloat16).reshape(
            1, CHUNK, D)

        sends2 = []
        for d in range(1, N_DEV):
            peer = lax.rem(me + d, N_DEV)
            rdma = pltpu.make_async_remote_copy(
                src_ref=gather_ref.at[me],
                dst_ref=gather_ref.at[me],
                send_sem=send2_sems.at[peer],
                recv_sem=recv2_sems.at[me],
                device_id=(peer,),
                device_id_type=pl.DeviceIdType.MESH,
            )
            rdma.start()
            sends2.append(rdma)
        for d in range(1, N_DEV):
            src = lax.rem(me + d, N_DEV)
            pltpu.make_async_remote_copy(
                src_ref=gather_ref.at[src],
                dst_ref=gather_ref.at[src],
                send_sem=send2_sems.at[src],
                recv_sem=recv2_sems.at[src],
                device_id=(src,),
                device_id_type=pl.DeviceIdType.MESH,
            ).wait_recv()
        for c in range(N_DEV):
            out_ref[c * CHUNK:(c + 1) * CHUNK, :] = (
                gather_ref[c].astype(jnp.float32))
        for rdma in sends2:
            rdma.wait_send()

    return pl.pallas_call(
        body,
        out_shape=jax.ShapeDtypeStruct((M, D), jnp.float32),
        in_specs=[pl.BlockSpec(memory_space=pltpu.VMEM)] * 4,
        out_specs=pl.BlockSpec(memory_space=pltpu.VMEM),
        scratch_shapes=[
            pltpu.VMEM((N_DEV, CHUNK, D), jnp.bfloat16),
            pltpu.VMEM((N_DEV, CHUNK, D), jnp.bfloat16),
            pltpu.VMEM((N_DEV, CHUNK, D), jnp.bfloat16),
            pltpu.SemaphoreType.DMA((N_DEV,)),
            pltpu.SemaphoreType.DMA((N_DEV,)),
            pltpu.SemaphoreType.DMA((N_DEV,)),
            pltpu.SemaphoreType.DMA((N_DEV,)),
        ],
        compiler_params=pltpu.CompilerParams(collective_id=0),
    )(x, Wg, Wu, Wd)


# device time: 50048 ns/iter; 1.1757x vs baseline; 1.1757x over previous
import jax
import jax.numpy as jnp
from jax import lax
from jax.experimental import pallas as pl
from jax.experimental.pallas import tpu as pltpu

N_DEV = 32
M = 768
D = 768
CHUNK = M // N_DEV
NH = 2
HALF = D // NH


def kernel(x, Wg, Wu, Wd):
    def body(x_ref, wg_ref, wu_ref, wd_ref, out_ref,
             partial_ref, comm_ref, gather_ref,
             send1_sems, recv1_sems, send2_sems, recv2_sems):
        me = lax.axis_index("i")
        my_rows = pl.ds(me * CHUNK, CHUNK)

        barrier = pltpu.get_barrier_semaphore()
        for d in range(1, N_DEV):
            peer = lax.rem(me + d, N_DEV)
            pl.semaphore_signal(barrier, inc=1, device_id=(peer,),
                                device_id_type=pl.DeviceIdType.MESH)

        xb = x_ref[...].astype(jnp.bfloat16)
        gate = jnp.dot(xb, wg_ref[...].astype(jnp.bfloat16),
                       preferred_element_type=jnp.float32)
        up = jnp.dot(xb, wu_ref[...].astype(jnp.bfloat16),
                     preferred_element_type=jnp.float32)
        act = (gate * (up * jax.nn.sigmoid(up))).astype(jnp.bfloat16)
        partial_ref[...] = jnp.dot(act, wd_ref[...].astype(jnp.bfloat16),
                                   preferred_element_type=jnp.float32
                                   ).astype(jnp.bfloat16)

        pl.semaphore_wait(barrier, N_DEV - 1)

        sends1 = []
        for h in range(NH):
            cols = pl.ds(h * HALF, HALF)
            for d in range(1, N_DEV):
                peer = lax.rem(me + d, N_DEV)
                rdma = pltpu.make_async_remote_copy(
                    src_ref=partial_ref.at[pl.ds(peer * CHUNK, CHUNK), cols],
                    dst_ref=comm_ref.at[my_rows, cols],
                    send_sem=send1_sems.at[h, peer],
                    recv_sem=recv1_sems.at[h, me],
                    device_id=(peer,),
                    device_id_type=pl.DeviceIdType.MESH,
                )
                rdma.start()
                sends1.append(rdma)

        comm_ref[my_rows, :] = partial_ref[my_rows, :]

        sends2 = []
        for h in range(NH):
            cols = pl.ds(h * HALF, HALF)
            for d in range(1, N_DEV):
                src = lax.rem(me + d, N_DEV)
                pltpu.make_async_remote_copy(
                    src_ref=partial_ref.at[my_rows, cols],
                    dst_ref=comm_ref.at[pl.ds(src * CHUNK, CHUNK), cols],
                    send_sem=send1_sems.at[h, src],
                    recv_sem=recv1_sems.at[h, src],
                    device_id=(src,),
                    device_id_type=pl.DeviceIdType.MESH,
                ).wait_recv()

            stacked = comm_ref[:, h * HALF:(h + 1) * HALF].reshape(
                N_DEV, CHUNK, HALF).astype(jnp.float32)
            reduced = jnp.sum(stacked, axis=0)
            gather_ref[my_rows, cols] = reduced.astype(jnp.bfloat16)

            for d in range(1, N_DEV):
                peer = lax.rem(me + d, N_DEV)
                rdma = pltpu.make_async_remote_copy(
                    src_ref=gather_ref.at[my_rows, cols],
                    dst_ref=gather_ref.at[my_rows, cols],
                    send_sem=send2_sems.at[h, peer],
                    recv_sem=recv2_sems.at[h, me],
                    device_id=(peer,),
                    device_id_type=pl.DeviceIdType.MESH,
                )
                rdma.start()
                sends2.append(rdma)

        for h in range(NH):
            cols = pl.ds(h * HALF, HALF)
            for d in range(1, N_DEV):
                src = lax.rem(me + d, N_DEV)
                pltpu.make_async_remote_copy(
                    src_ref=gather_ref.at[my_rows, cols],
                    dst_ref=gather_ref.at[pl.ds(src * CHUNK, CHUNK), cols],
                    send_sem=send2_sems.at[h, src],
                    recv_sem=recv2_sems.at[h, src],
                    device_id=(src,),
                    device_id_type=pl.DeviceIdType.MESH,
                ).wait_recv()
        out_ref[...] = gather_ref[...].astype(jnp.float32)
        for rdma in sends1:
            rdma.wait_send()
        for rdma in sends2:
            rdma.wait_send()

    return pl.pallas_call(
        body,
        out_shape=jax.ShapeDtypeStruct((M, D), jnp.float32),
        in_specs=[pl.BlockSpec(memory_space=pltpu.VMEM)] * 4,
        out_specs=pl.BlockSpec(memory_space=pltpu.VMEM),
        scratch_shapes=[
            pltpu.VMEM((M, D), jnp.bfloat16),
            pltpu.VMEM((M, D), jnp.bfloat16),
            pltpu.VMEM((M, D), jnp.bfloat16),
            pltpu.SemaphoreType.DMA((NH, N_DEV)),
            pltpu.SemaphoreType.DMA((NH, N_DEV)),
            pltpu.SemaphoreType.DMA((NH, N_DEV)),
            pltpu.SemaphoreType.DMA((NH, N_DEV)),
        ],
        compiler_params=pltpu.CompilerParams(collective_id=0),
    )(x, Wg, Wu, Wd)
